# baseline (speedup 1.0000x reference)
import functools

import jax
import jax.numpy as jnp
from jax import lax
from jax.experimental import pallas as pl
from jax.experimental.pallas import tpu as pltpu
from jax.experimental.pallas import tpu_sc as plsc

N = 100000
D = 128
S = 512
L = 16
NG = D // L
NC = 2
NS = 16
NW = NC * NS
CL = 128
K = 25
T = -(-N // CL)
NBUF = 7
SROWS = S // NS


def _sc_partial_sums(x, batch):
  mesh = plsc.VectorSubcoreMesh(core_axis_name="c", subcore_axis_name="s")

  @functools.partial(
      pl.kernel,
      out_type=jax.ShapeDtypeStruct((NC, S, D), jnp.float32),
      mesh=mesh,
      scratch_types=[
          pltpu.VMEM((K, CL), jnp.int32),
          [pltpu.VMEM((CL, D), jnp.float32) for _ in range(NBUF)],
          pltpu.VMEM_SHARED((S + 1, D), jnp.float32),
          pltpu.SemaphoreType.DMA,
          [pltpu.SemaphoreType.DMA for _ in range(NBUF)],
          [pltpu.SemaphoreType.DMA for _ in range(NBUF)],
      ],
  )
  def k(x_hbm, b_hbm, out_hbm, ids_v, bufs, acc_sh, sid_sem,
        ssems, csems):
    cid = lax.axis_index("c")
    sid = lax.axis_index("s")
    wid = cid * NS + sid
    zero = jnp.zeros((L,), jnp.float32)
    lanes = lax.iota(jnp.int32, L)

    def chunk_start(j):
      return pl.multiple_of(jnp.minimum((wid * K + j) * CL, N - CL), 8)

    for j in range(K):
      pltpu.async_copy(
          b_hbm.at[pl.ds(chunk_start(j), CL)], ids_v.at[j], sid_sem)

    def zrow(i, carry):
      for j in range(NG):
        bufs[0].at[i][pl.ds(j * L, L)] = zero
      return carry
    lax.fori_loop(0, SROWS, zrow, 0)
    pltpu.sync_copy(bufs[0].at[pl.ds(0, SROWS)],
                    acc_sh.at[pl.ds(sid * SROWS, SROWS)])
    plsc.subcore_barrier()

    for j in range(K):
      pltpu.make_async_copy(
          b_hbm.at[pl.ds(0, CL)], ids_v.at[j], sid_sem).wait()

    def mask_row(j):
      c = wid * K + j
      st = chunk_start(j)
      for g in range(CL // L):
        gl = st + g * L + lanes
        v = ids_v.at[j][pl.ds(g * L, L)]
        ids_v.at[j][pl.ds(g * L, L)] = jnp.where(gl >= c * CL, v, S)

    def issue_stage(j):
      b = j % NBUF
      pltpu.async_copy(
          x_hbm.at[pl.ds(chunk_start(j), CL)], bufs[b], ssems[b])

    def wait_stage(j):
      b = j % NBUF
      pltpu.make_async_copy(
          x_hbm.at[pl.ds(0, CL)], bufs[b], ssems[b]).wait()

    def issue_scatter(j):
      b = j % NBUF
      pltpu.async_copy(bufs[b], acc_sh.at[ids_v.at[j]], csems[b], add=True)

    def wait_scatter(j):
      b = j % NBUF
      pltpu.make_async_copy(
          bufs[b], acc_sh.at[ids_v.at[j]], csems[b]).wait()

    LOOK = NBUF // 2
    for j in range(min(2 * LOOK, K)):
      issue_stage(j)
    for j in range(K):
      wait_stage(j)
      mask_row(j)
      issue_scatter(j)
      if j - LOOK >= 0 and j + LOOK < K:
        wait_scatter(j - LOOK)
        issue_stage(j + LOOK)
    for j in range(max(K - 2 * LOOK, 0), K):
      wait_scatter(j)

    plsc.subcore_barrier()
    pltpu.sync_copy(acc_sh.at[pl.ds(sid * SROWS, SROWS)],
                    bufs[0].at[pl.ds(0, SROWS)])
    pltpu.sync_copy(bufs[0].at[pl.ds(0, SROWS)],
                    out_hbm.at[cid, pl.ds(sid * SROWS, SROWS)])

  return k(x, batch)


def _combine(parts):
  def body(p_ref, o_ref):
    o_ref[...] = p_ref[0] + p_ref[1]

  return pl.pallas_call(
      body,
      out_shape=jax.ShapeDtypeStruct((S, D), jnp.float32),
  )(parts)


@jax.jit
def kernel(x, batch):
  parts = _sc_partial_sums(x, batch.astype(jnp.int32))
  return _combine(parts)


# speedup vs baseline: 1.0583x; 1.0583x over previous
import functools

import jax
import jax.numpy as jnp
from jax import lax
from jax.experimental import pallas as pl
from jax.experimental.pallas import tpu as pltpu
from jax.experimental.pallas import tpu_sc as plsc

N = 100000
D = 128
S = 512
L = 16
NG = D // L
NC = 2
NS = 16
NW = NC * NS
CL = 128
K = 25
T = -(-N // CL)
NBUF = 7
SROWS = S // NS


def _sc_partial_sums(x, batch):
  mesh = plsc.VectorSubcoreMesh(core_axis_name="c", subcore_axis_name="s")

  @functools.partial(
      pl.kernel,
      out_type=jax.ShapeDtypeStruct((NC, S, D), jnp.float32),
      mesh=mesh,
      scratch_types=[
          pltpu.VMEM((K, CL), jnp.int32),
          [pltpu.VMEM((CL, D), jnp.float32) for _ in range(NBUF)],
          pltpu.VMEM_SHARED((S + 1, D), jnp.float32),
          pltpu.SemaphoreType.DMA,
          [pltpu.SemaphoreType.DMA for _ in range(NBUF)],
          [pltpu.SemaphoreType.DMA for _ in range(NBUF)],
      ],
  )
  def k(x_hbm, b_hbm, out_hbm, ids_v, bufs, acc_sh, sid_sem,
        ssems, csems):
    cid = lax.axis_index("c")
    sid = lax.axis_index("s")
    wid = cid * NS + sid
    zero = jnp.zeros((L,), jnp.float32)
    lanes = lax.iota(jnp.int32, L)

    def chunk_start(j):
      return pl.multiple_of(jnp.minimum((wid + NW * j) * CL, N - CL), 8)

    for j in range(K):
      pltpu.async_copy(
          b_hbm.at[pl.ds(chunk_start(j), CL)], ids_v.at[j], sid_sem)

    def zrow(i, carry):
      for j in range(NG):
        bufs[0].at[i][pl.ds(j * L, L)] = zero
      return carry
    lax.fori_loop(0, SROWS, zrow, 0)
    pltpu.sync_copy(bufs[0].at[pl.ds(0, SROWS)],
                    acc_sh.at[pl.ds(sid * SROWS, SROWS)])
    plsc.subcore_barrier()

    for j in range(K):
      pltpu.make_async_copy(
          b_hbm.at[pl.ds(0, CL)], ids_v.at[j], sid_sem).wait()

    def mask_row(j):
      c = wid + NW * j
      st = chunk_start(j)
      for g in range(CL // L):
        gl = st + g * L + lanes
        v = ids_v.at[j][pl.ds(g * L, L)]
        ids_v.at[j][pl.ds(g * L, L)] = jnp.where(gl >= c * CL, v, S)

    def issue_stage(j):
      b = j % NBUF
      pltpu.async_copy(
          x_hbm.at[pl.ds(chunk_start(j), CL)], bufs[b], ssems[b])

    def wait_stage(j):
      b = j % NBUF
      pltpu.make_async_copy(
          x_hbm.at[pl.ds(0, CL)], bufs[b], ssems[b]).wait()

    def issue_scatter(j):
      b = j % NBUF
      pltpu.async_copy(bufs[b], acc_sh.at[ids_v.at[j]], csems[b], add=True)

    def wait_scatter(j):
      b = j % NBUF
      pltpu.make_async_copy(
          bufs[b], acc_sh.at[ids_v.at[j]], csems[b]).wait()

    LOOK = NBUF // 2
    for j in range(min(2 * LOOK, K)):
      issue_stage(j)
    for j in range(K):
      wait_stage(j)
      mask_row(j)
      issue_scatter(j)
      if j - LOOK >= 0 and j + LOOK < K:
        wait_scatter(j - LOOK)
        issue_stage(j + LOOK)
    for j in range(max(K - 2 * LOOK, 0), K):
      wait_scatter(j)

    plsc.subcore_barrier()
    pltpu.sync_copy(acc_sh.at[pl.ds(sid * SROWS, SROWS)],
                    bufs[0].at[pl.ds(0, SROWS)])
    pltpu.sync_copy(bufs[0].at[pl.ds(0, SROWS)],
                    out_hbm.at[cid, pl.ds(sid * SROWS, SROWS)])

  return k(x, batch)


def _combine(parts):
  def body(p_ref, o_ref):
    o_ref[...] = p_ref[0] + p_ref[1]

  return pl.pallas_call(
      body,
      out_shape=jax.ShapeDtypeStruct((S, D), jnp.float32),
  )(parts)


@jax.jit
def kernel(x, batch):
  parts = _sc_partial_sums(x, batch.astype(jnp.int32))
  return _combine(parts)


# speedup vs baseline: 1.0895x; 1.0296x over previous
import functools

import jax
import jax.numpy as jnp
from jax import lax
from jax.experimental import pallas as pl
from jax.experimental.pallas import tpu as pltpu
from jax.experimental.pallas import tpu_sc as plsc

N = 100000
D = 128
S = 512
L = 16
NG = D // L
NC = 2
NS = 16
NW = NC * NS
CL = 128
K = 25
T = -(-N // CL)
NBUF = 7
SROWS = S // NS


def _sc_partial_sums(x, batch):
  mesh = plsc.VectorSubcoreMesh(core_axis_name="c", subcore_axis_name="s")

  @functools.partial(
      pl.kernel,
      out_type=jax.ShapeDtypeStruct((NC, S, D), jnp.float32),
      mesh=mesh,
      scratch_types=[
          pltpu.VMEM((K, CL), jnp.int32),
          [pltpu.VMEM((CL, D), jnp.float32) for _ in range(NBUF)],
          pltpu.VMEM_SHARED((S + 1, D), jnp.float32),
          pltpu.SemaphoreType.DMA,
          [pltpu.SemaphoreType.DMA for _ in range(NBUF)],
          [pltpu.SemaphoreType.DMA for _ in range(NBUF)],
      ],
  )
  def k(x_hbm, b_hbm, out_hbm, ids_v, bufs, acc_sh, sid_sem,
        ssems, csems):
    cid = lax.axis_index("c")
    sid = lax.axis_index("s")
    wid = cid * NS + sid
    zero = jnp.zeros((L,), jnp.float32)
    lanes = lax.iota(jnp.int32, L)

    def chunk_start(j):
      return pl.multiple_of(jnp.minimum((wid + NW * j) * CL, N - CL), 8)

    for j in range(K):
      pltpu.async_copy(
          b_hbm.at[pl.ds(chunk_start(j), CL)], ids_v.at[j], sid_sem)

    def zrow(i, carry):
      for j in range(NG):
        bufs[0].at[i][pl.ds(j * L, L)] = zero
      return carry
    lax.fori_loop(0, SROWS, zrow, 0)
    pltpu.sync_copy(bufs[0].at[pl.ds(0, SROWS)],
                    acc_sh.at[pl.ds(sid * SROWS, SROWS)])
    plsc.subcore_barrier()

    for j in range(K):
      pltpu.make_async_copy(
          b_hbm.at[pl.ds(0, CL)], ids_v.at[j], sid_sem).wait()

    def mask_row(j):
      c = wid + NW * j
      st = chunk_start(j)
      @pl.when(st != c * CL)
      def _():
        for g in range(CL // L):
          gl = st + g * L + lanes
          v = ids_v.at[j][pl.ds(g * L, L)]
          ids_v.at[j][pl.ds(g * L, L)] = jnp.where(gl >= c * CL, v, S)

    def issue_stage(j):
      b = j % NBUF
      pltpu.async_copy(
          x_hbm.at[pl.ds(chunk_start(j), CL)], bufs[b], ssems[b])

    def wait_stage(j):
      b = j % NBUF
      pltpu.make_async_copy(
          x_hbm.at[pl.ds(0, CL)], bufs[b], ssems[b]).wait()

    def issue_scatter(j):
      b = j % NBUF
      pltpu.async_copy(bufs[b], acc_sh.at[ids_v.at[j]], csems[b], add=True)

    def wait_scatter(j):
      b = j % NBUF
      pltpu.make_async_copy(
          bufs[b], acc_sh.at[ids_v.at[j]], csems[b]).wait()

    LOOK = NBUF // 2
    for j in range(min(2 * LOOK, K)):
      issue_stage(j)
    for j in range(K):
      wait_stage(j)
      mask_row(j)
      issue_scatter(j)
      if j - LOOK >= 0 and j + LOOK < K:
        wait_scatter(j - LOOK)
        issue_stage(j + LOOK)
    for j in range(max(K - 2 * LOOK, 0), K):
      wait_scatter(j)

    plsc.subcore_barrier()
    pltpu.sync_copy(acc_sh.at[pl.ds(sid * SROWS, SROWS)],
                    bufs[0].at[pl.ds(0, SROWS)])
    pltpu.sync_copy(bufs[0].at[pl.ds(0, SROWS)],
                    out_hbm.at[cid, pl.ds(sid * SROWS, SROWS)])

  return k(x, batch)


def _combine(parts):
  def body(p_ref, o_ref):
    o_ref[...] = p_ref[0] + p_ref[1]

  return pl.pallas_call(
      body,
      out_shape=jax.ShapeDtypeStruct((S, D), jnp.float32),
  )(parts)


@jax.jit
def kernel(x, batch):
  parts = _sc_partial_sums(x, batch.astype(jnp.int32))
  return _combine(parts)
